# baseline (speedup 1.0000x reference)
import functools

import jax
import jax.numpy as jnp
from jax import lax
from jax.experimental import pallas as pl
from jax.experimental.pallas import tpu as pltpu
from jax.experimental.pallas import tpu_sc as plsc

_BATCH = 16384
_D = 64
_LANES = 16
_NC = 2
_NS = 16
_NW = _NC * _NS
_BPW = _BATCH // _NW
_NCLS = 1000000
_TCH = 31744


def _tc_transpose(centers_t):
    def body(x_ref, o_ref):
        o_ref[...] = x_ref[...].T

    half = 16
    return pl.pallas_call(
        body,
        grid=(2, half),
        in_specs=[pl.BlockSpec((_D, _TCH), lambda c, i: (0, c * half + i))],
        out_specs=pl.BlockSpec((_TCH, _D), lambda c, i: (c * half + i, 0)),
        out_shape=jax.ShapeDtypeStruct((_NCLS, _D), jnp.float32),
        compiler_params=pltpu.CompilerParams(
            dimension_semantics=("parallel", "arbitrary")),
    )(centers_t)


def _sc_partials(features_t, labels, centers_rm):
    mesh = plsc.VectorSubcoreMesh(core_axis_name="c", subcore_axis_name="s")

    @functools.partial(
        pl.kernel,
        out_type=jax.ShapeDtypeStruct((_NW, _LANES), jnp.float32),
        mesh=mesh,
        compiler_params=pltpu.CompilerParams(needs_layout_passes=False),
        scratch_types=[
            pltpu.VMEM((_BPW,), jnp.int32),
            pltpu.VMEM((_D, _BPW), jnp.float32),
            pltpu.VMEM((_BPW, _D), jnp.float32),
            pltpu.VMEM((_LANES,), jnp.float32),
            pltpu.SemaphoreType.DMA,
            pltpu.SemaphoreType.DMA,
        ],
    )
    def k(f_hbm, l_hbm, c_hbm, out_hbm, idx_v, f_v, g2_v, acc_v, gsem, fsem):
        wid = lax.axis_index("s") * _NC + lax.axis_index("c")
        base = wid * _BPW
        pltpu.sync_copy(l_hbm.at[pl.ds(base, _BPW)], idx_v)
        fcp = pltpu.async_copy(f_hbm.at[:, pl.ds(base, _BPW)], f_v, fsem)

        @pl.loop(0, _BPW, step=_LANES)
        def _(r):
            iv = idx_v[pl.ds(r, _LANES)]
            for j in range(_LANES):
                pltpu.async_copy(
                    c_hbm.at[pl.ds(iv[j], 1)], g2_v.at[pl.ds(r + j, 1)], gsem)

        pltpu.make_async_copy(c_hbm.at[pl.ds(0, _BPW)], g2_v, gsem).wait()
        fcp.wait()

        zero = jnp.zeros((_LANES,), jnp.float32)
        drange = lax.iota(jnp.int32, _LANES)

        @pl.loop(0, _BPW, init_carry=(zero, zero, zero, zero), unroll=2)
        def acc(r, carry):
            a0, a1, a2, a3 = carry
            col = jnp.full((_LANES,), r, jnp.int32)
            f0 = plsc.load_gather(f_v, [drange, col])
            f1 = plsc.load_gather(f_v, [drange + 16, col])
            f2 = plsc.load_gather(f_v, [drange + 32, col])
            f3 = plsc.load_gather(f_v, [drange + 48, col])
            d0 = f0 - g2_v[r, pl.ds(0, _LANES)]
            d1 = f1 - g2_v[r, pl.ds(16, _LANES)]
            d2 = f2 - g2_v[r, pl.ds(32, _LANES)]
            d3 = f3 - g2_v[r, pl.ds(48, _LANES)]
            return (a0 + d0 * d0, a1 + d1 * d1, a2 + d2 * d2, a3 + d3 * d3)

        a0, a1, a2, a3 = acc
        acc_v[...] = (a0 + a1) + (a2 + a3)
        pltpu.sync_copy(acc_v, out_hbm.at[wid])

    return k(features_t, labels, centers_rm)


def _tc_reduce(partials):
    def body(p_ref, o_ref):
        o_ref[...] = jnp.sum(p_ref[...], keepdims=True).reshape(1, 1) * (0.5 / _BATCH)

    out = pl.pallas_call(
        body,
        out_shape=jax.ShapeDtypeStruct((1, 1), jnp.float32),
    )(partials)
    return out[0, 0]


def kernel(features, labels, centers):
    labels_i = labels.astype(jnp.int32)
    centers_rm = _tc_transpose(centers.T)
    partials = _sc_partials(features.T, labels_i, centers_rm)
    return _tc_reduce(partials)


# speedup vs baseline: 1.0071x; 1.0071x over previous
import functools

import jax
import jax.numpy as jnp
from jax import lax
from jax.experimental import pallas as pl
from jax.experimental.pallas import tpu as pltpu
from jax.experimental.pallas import tpu_sc as plsc

_BATCH = 16384
_D = 64
_LANES = 16
_NC = 2
_NS = 16
_NW = _NC * _NS
_BPW = _BATCH // _NW
_FPW = _BPW * _D
_NCLS = 1000000
_TCH = 31744


def _tc_transpose(centers_t):
    def body(x_ref, o_ref):
        o_ref[...] = x_ref[...].T

    half = 16
    return pl.pallas_call(
        body,
        grid=(2, half),
        in_specs=[pl.BlockSpec((_D, _TCH), lambda c, i: (0, c * half + i))],
        out_specs=pl.BlockSpec((_TCH, _D), lambda c, i: (c * half + i, 0)),
        out_shape=jax.ShapeDtypeStruct((_NCLS, _D), jnp.float32),
        compiler_params=pltpu.CompilerParams(
            dimension_semantics=("parallel", "arbitrary")),
    )(centers_t)


def _sc_partials(features_flat, labels, centers_rm):
    mesh = plsc.VectorSubcoreMesh(core_axis_name="c", subcore_axis_name="s")

    @functools.partial(
        pl.kernel,
        out_type=jax.ShapeDtypeStruct((_NW, _LANES), jnp.float32),
        mesh=mesh,
        scratch_types=[
            pltpu.VMEM((_BPW,), jnp.int32),
            pltpu.VMEM((_FPW,), jnp.float32),
            pltpu.VMEM((_BPW, _D), jnp.float32),
            pltpu.VMEM((_LANES,), jnp.float32),
            pltpu.SemaphoreType.DMA,
            pltpu.SemaphoreType.DMA,
        ],
    )
    def k(f_hbm, l_hbm, c_hbm, out_hbm, idx_v, f_v, g2_v, acc_v, gsem, fsem):
        wid = lax.axis_index("s") * _NC + lax.axis_index("c")
        base = wid * _BPW
        pltpu.sync_copy(l_hbm.at[pl.ds(base, _BPW)], idx_v)
        fcp = pltpu.async_copy(f_hbm.at[pl.ds(base * _D, _FPW)], f_v, fsem)

        @pl.loop(0, _BPW, step=_LANES)
        def _(r):
            iv = idx_v[pl.ds(r, _LANES)]
            for j in range(_LANES):
                pltpu.async_copy(
                    c_hbm.at[pl.ds(iv[j], 1)], g2_v.at[pl.ds(r + j, 1)], gsem)

        pltpu.make_async_copy(c_hbm.at[pl.ds(0, _BPW)], g2_v, gsem).wait()
        fcp.wait()

        zero = jnp.zeros((_LANES,), jnp.float32)

        @pl.loop(0, _BPW, init_carry=(zero, zero, zero, zero), unroll=2)
        def acc(r, carry):
            a0, a1, a2, a3 = carry
            f = r * _D
            d0 = f_v[pl.ds(f, _LANES)] - g2_v[r, pl.ds(0, _LANES)]
            d1 = f_v[pl.ds(f + 16, _LANES)] - g2_v[r, pl.ds(16, _LANES)]
            d2 = f_v[pl.ds(f + 32, _LANES)] - g2_v[r, pl.ds(32, _LANES)]
            d3 = f_v[pl.ds(f + 48, _LANES)] - g2_v[r, pl.ds(48, _LANES)]
            return (a0 + d0 * d0, a1 + d1 * d1, a2 + d2 * d2, a3 + d3 * d3)

        a0, a1, a2, a3 = acc
        acc_v[...] = (a0 + a1) + (a2 + a3)
        pltpu.sync_copy(acc_v, out_hbm.at[wid])

    return k(features_flat, labels, centers_rm)


def _tc_reduce(partials):
    def body(p_ref, o_ref):
        o_ref[...] = jnp.sum(p_ref[...], keepdims=True).reshape(1, 1) * (0.5 / _BATCH)

    out = pl.pallas_call(
        body,
        out_shape=jax.ShapeDtypeStruct((1, 1), jnp.float32),
    )(partials)
    return out[0, 0]


def kernel(features, labels, centers):
    labels_i = labels.astype(jnp.int32)
    centers_rm = _tc_transpose(centers.T)
    partials = _sc_partials(features.reshape(-1), labels_i, centers_rm)
    return _tc_reduce(partials)
